# baseline (speedup 1.0000x reference)
import jax
import jax.numpy as jnp
from jax import lax
from jax.experimental import pallas as pl
from jax.experimental.pallas import tpu as pltpu
from jax.experimental.pallas import tpu_sc as plsc

B, S = 4096, 200
D = 32
NC, NS = 2, 16
NW = NC * NS
BW = B // NW
LG = BW // 16


def _emb_body(ids_hbm, wre_hbm, wim_hbm, zre_hbm, zim_hbm,
              idx_v, gb_re, gb_im, tb_re, tb_im,
              gs0, gs1, ws0, ws1):
    gsems = (gs0, gs1)
    wsems = (ws0, ws1)
    wid = lax.axis_index("s") * NC + lax.axis_index("c")
    bw0 = wid * BW

    pltpu.sync_copy(ids_hbm.at[:, pl.ds(bw0, BW)], idx_v)

    iota = lax.iota(jnp.int32, 16)

    def g_copies(s, j, mk):
        return [mk(wre_hbm.at[idx_v.at[s]], gb_re.at[j], gsems[j]),
                mk(wim_hbm.at[idx_v.at[s]], gb_im.at[j], gsems[j])]

    def w_copies(s, j, mk):
        return [mk(tb_re.at[j], zre_hbm.at[s, :, pl.ds(bw0, BW)], wsems[j]),
                mk(tb_im.at[j], zim_hbm.at[s, :, pl.ds(bw0, BW)], wsems[j])]

    def transpose(j):
        slotv = jnp.full((16,), j, jnp.int32)

        def drow(d, carry):
            dv = jnp.full((16,), 0, jnp.int32) + d
            for b0 in range(LG):
                bv = iota + (16 * b0)
                tb_re[j, d, pl.ds(16 * b0, 16)] = plsc.load_gather(
                    gb_re, [slotv, bv, dv])
                tb_im[j, d, pl.ds(16 * b0, 16)] = plsc.load_gather(
                    gb_im, [slotv, bv, dv])
            return carry

        lax.fori_loop(0, D, drow, 0)

    def step(s, j, *, first, last):
        for d in g_copies(s, j, pltpu.make_async_copy):
            d.wait()
        if not last:
            g_copies(s + 1, 1 - j, pltpu.async_copy)
        if not first:
            for d in w_copies(s - 2, j, pltpu.make_async_copy):
                d.wait()
        transpose(j)
        w_copies(s, j, pltpu.async_copy)

    g_copies(0, 0, pltpu.async_copy)
    step(0, 0, first=True, last=False)
    step(1, 1, first=True, last=False)

    def steady(p, carry):
        s0 = 2 * p
        step(s0, 0, first=False, last=False)
        step(s0 + 1, 1, first=False, last=False)
        return carry

    lax.fori_loop(1, (S - 2) // 2, steady, 0)

    step(S - 2, 0, first=False, last=False)
    step(S - 1, 1, first=False, last=True)
    for d in w_copies(S - 2, 0, pltpu.make_async_copy):
        d.wait()
    for d in w_copies(S - 1, 1, pltpu.make_async_copy):
        d.wait()


@jax.jit
def kernel(input_ids, W_re, W_im):
    ids_t = input_ids.T
    mesh = plsc.VectorSubcoreMesh(core_axis_name="c", subcore_axis_name="s")
    z_re, z_im = pl.kernel(
        _emb_body,
        out_type=[
            jax.ShapeDtypeStruct((S, D, B), jnp.float32),
            jax.ShapeDtypeStruct((S, D, B), jnp.float32),
        ],
        mesh=mesh,
        scratch_types=[
            pltpu.VMEM((S, BW), jnp.int32),
            pltpu.VMEM((2, BW, D), jnp.float32),
            pltpu.VMEM((2, BW, D), jnp.float32),
            pltpu.VMEM((2, D, BW), jnp.float32),
            pltpu.VMEM((2, D, BW), jnp.float32),
        ] + [pltpu.SemaphoreType.DMA] * 4,
        compiler_params=pltpu.CompilerParams(
            use_tc_tiling_on_sc=False, needs_layout_passes=False),
    )(ids_t, W_re, W_im)
    return (z_re.transpose(2, 0, 1), z_im.transpose(2, 0, 1))


# speedup vs baseline: 1.1246x; 1.1246x over previous
import jax
import jax.numpy as jnp
from jax import lax
from jax.experimental import pallas as pl
from jax.experimental.pallas import tpu as pltpu
from jax.experimental.pallas import tpu_sc as plsc

B, S = 4096, 200
D = 32
NC, NS = 2, 16
NW = NC * NS
BW = B // NW
LG = BW // 16


def _emb_body(ids_hbm, wre_hbm, wim_hbm, zre_hbm, zim_hbm,
              idx_v, gb_re, gb_im, tb_re, tb_im,
              gs0, gs1, ws0, ws1):
    gsems = (gs0, gs1)
    wsems = (ws0, ws1)
    wid = lax.axis_index("s") * NC + lax.axis_index("c")
    bw0 = wid * BW

    pltpu.sync_copy(ids_hbm.at[:, pl.ds(bw0, BW)], idx_v)

    iota = lax.iota(jnp.int32, 16)

    def g_copies(s, j, mk):
        return [mk(wre_hbm.at[idx_v.at[s]], gb_re.at[j], gsems[j]),
                mk(wim_hbm.at[idx_v.at[s]], gb_im.at[j], gsems[j])]

    def w_copies(s, j, mk):
        return [mk(tb_re.at[j], zre_hbm.at[s, :, pl.ds(bw0, BW)], wsems[j]),
                mk(tb_im.at[j], zim_hbm.at[s, :, pl.ds(bw0, BW)], wsems[j])]

    def transpose(j):
        slotv = jnp.full((16,), j, jnp.int32)
        dlo = iota
        dhi = iota + 16

        def brow(b, carry):
            bv = jnp.full((16,), 0, jnp.int32) + b
            v0 = gb_re[j, b, pl.ds(0, 16)]
            v1 = gb_re[j, b, pl.ds(16, 16)]
            plsc.store_scatter(tb_re, [slotv, dlo, bv], v0)
            plsc.store_scatter(tb_re, [slotv, dhi, bv], v1)
            w0 = gb_im[j, b, pl.ds(0, 16)]
            w1 = gb_im[j, b, pl.ds(16, 16)]
            plsc.store_scatter(tb_im, [slotv, dlo, bv], w0)
            plsc.store_scatter(tb_im, [slotv, dhi, bv], w1)
            return carry

        lax.fori_loop(0, BW, brow, 0, unroll=8)

    def step(s, j, *, first, last):
        for d in g_copies(s, j, pltpu.make_async_copy):
            d.wait()
        if not last:
            g_copies(s + 1, 1 - j, pltpu.async_copy)
        if not first:
            for d in w_copies(s - 2, j, pltpu.make_async_copy):
                d.wait()
        transpose(j)
        w_copies(s, j, pltpu.async_copy)

    g_copies(0, 0, pltpu.async_copy)
    step(0, 0, first=True, last=False)
    step(1, 1, first=True, last=False)

    def steady(p, carry):
        s0 = 2 * p
        step(s0, 0, first=False, last=False)
        step(s0 + 1, 1, first=False, last=False)
        return carry

    lax.fori_loop(1, (S - 2) // 2, steady, 0)

    step(S - 2, 0, first=False, last=False)
    step(S - 1, 1, first=False, last=True)
    for d in w_copies(S - 2, 0, pltpu.make_async_copy):
        d.wait()
    for d in w_copies(S - 1, 1, pltpu.make_async_copy):
        d.wait()


@jax.jit
def kernel(input_ids, W_re, W_im):
    ids_t = input_ids.T
    mesh = plsc.VectorSubcoreMesh(core_axis_name="c", subcore_axis_name="s")
    z_re, z_im = pl.kernel(
        _emb_body,
        out_type=[
            jax.ShapeDtypeStruct((S, D, B), jnp.float32),
            jax.ShapeDtypeStruct((S, D, B), jnp.float32),
        ],
        mesh=mesh,
        scratch_types=[
            pltpu.VMEM((S, BW), jnp.int32),
            pltpu.VMEM((2, BW, D), jnp.float32),
            pltpu.VMEM((2, BW, D), jnp.float32),
            pltpu.VMEM((2, D, BW), jnp.float32),
            pltpu.VMEM((2, D, BW), jnp.float32),
        ] + [pltpu.SemaphoreType.DMA] * 4,
        compiler_params=pltpu.CompilerParams(
            use_tc_tiling_on_sc=False, needs_layout_passes=False),
    )(ids_t, W_re, W_im)
    return (z_re.transpose(2, 0, 1), z_im.transpose(2, 0, 1))


# speedup vs baseline: 1.6407x; 1.4589x over previous
import jax
import jax.numpy as jnp
from jax import lax
from jax.experimental import pallas as pl
from jax.experimental.pallas import tpu as pltpu
from jax.experimental.pallas import tpu_sc as plsc

B, S = 4096, 200
D = 32
NC, NS = 2, 16
NW = NC * NS
BW = B // NW
LG = BW // 16


def _emb_body(ids_hbm, wre_hbm, wim_hbm, zre_hbm, zim_hbm,
              idx_v, gb_re, gb_im, tb_re, tb_im,
              gs0, gs1, ws0, ws1):
    gsems = (gs0, gs1)
    wsems = (ws0, ws1)
    wid = lax.axis_index("s") * NC + lax.axis_index("c")
    bw0 = wid * BW

    pltpu.sync_copy(ids_hbm.at[:, pl.ds(bw0, BW)], idx_v)

    iota = lax.iota(jnp.int32, 16)

    def g_copies(s, j, mk):
        return [mk(wre_hbm.at[idx_v.at[s]], gb_re.at[j], gsems[j]),
                mk(wim_hbm.at[idx_v.at[s]], gb_im.at[j], gsems[j])]

    def w_copies(s, j, mk):
        return [mk(tb_re.at[j, :, pl.ds(0, BW)],
                   zre_hbm.at[s, :, pl.ds(bw0, BW)], wsems[j]),
                mk(tb_im.at[j, :, pl.ds(0, BW)],
                   zim_hbm.at[s, :, pl.ds(bw0, BW)], wsems[j])]

    def transpose(j):
        slotv = jnp.full((16,), j, jnp.int32)
        dlo = iota
        dhi = iota + 16

        def brow(b, carry):
            bv = jnp.full((16,), 0, jnp.int32) + b
            v0 = gb_re[j, b, pl.ds(0, 16)]
            v1 = gb_re[j, b, pl.ds(16, 16)]
            plsc.store_scatter(tb_re, [slotv, dlo, bv], v0)
            plsc.store_scatter(tb_re, [slotv, dhi, bv], v1)
            w0 = gb_im[j, b, pl.ds(0, 16)]
            w1 = gb_im[j, b, pl.ds(16, 16)]
            plsc.store_scatter(tb_im, [slotv, dlo, bv], w0)
            plsc.store_scatter(tb_im, [slotv, dhi, bv], w1)
            return carry

        lax.fori_loop(0, BW, brow, 0, unroll=8)

    def step(s, j, *, first, last):
        for d in g_copies(s, j, pltpu.make_async_copy):
            d.wait()
        if not last:
            g_copies(s + 1, 1 - j, pltpu.async_copy)
        if not first:
            for d in w_copies(s - 2, j, pltpu.make_async_copy):
                d.wait()
        transpose(j)
        w_copies(s, j, pltpu.async_copy)

    g_copies(0, 0, pltpu.async_copy)
    step(0, 0, first=True, last=False)
    step(1, 1, first=True, last=False)

    def steady(p, carry):
        s0 = 2 * p
        step(s0, 0, first=False, last=False)
        step(s0 + 1, 1, first=False, last=False)
        return carry

    lax.fori_loop(1, (S - 2) // 2, steady, 0)

    step(S - 2, 0, first=False, last=False)
    step(S - 1, 1, first=False, last=True)
    for d in w_copies(S - 2, 0, pltpu.make_async_copy):
        d.wait()
    for d in w_copies(S - 1, 1, pltpu.make_async_copy):
        d.wait()


@jax.jit
def kernel(input_ids, W_re, W_im):
    ids_t = input_ids.T
    mesh = plsc.VectorSubcoreMesh(core_axis_name="c", subcore_axis_name="s")
    z_re, z_im = pl.kernel(
        _emb_body,
        out_type=[
            jax.ShapeDtypeStruct((S, D, B), jnp.float32),
            jax.ShapeDtypeStruct((S, D, B), jnp.float32),
        ],
        mesh=mesh,
        scratch_types=[
            pltpu.VMEM((S, BW), jnp.int32),
            pltpu.VMEM((2, BW, D), jnp.float32),
            pltpu.VMEM((2, BW, D), jnp.float32),
            pltpu.VMEM((2, D, BW + 1), jnp.float32),
            pltpu.VMEM((2, D, BW + 1), jnp.float32),
        ] + [pltpu.SemaphoreType.DMA] * 4,
        compiler_params=pltpu.CompilerParams(
            use_tc_tiling_on_sc=False, needs_layout_passes=False),
    )(ids_t, W_re, W_im)
    return (z_re.transpose(2, 0, 1), z_im.transpose(2, 0, 1))
